# Initial kernel scaffold; baseline (speedup 1.0000x reference)
import jax
import jax.numpy as jnp
from jax.experimental import pallas as pl


def kernel(prediction):
    raise NotImplementedError("write your pallas kernel here")


# speedup vs baseline: 397.8803x; 397.8803x over previous
import functools

import jax
import jax.numpy as jnp
from jax.experimental import pallas as pl

_CONF_THRES = 0.1
_IOU_THRES = 0.6
_MAX_WH = 4096.0
_MAX_DET = 300
_W = 128
_OUT_H = 304


def _nms_kernel(ch_ref, out_ref, *, n, nc, h):
    f32 = jnp.float32
    neg_inf = f32(-jnp.inf)
    cx = ch_ref[0]
    cy = ch_ref[1]
    w = ch_ref[2]
    hh = ch_ref[3]
    obj = ch_ref[4]
    x1 = cx - w / 2
    y1 = cy - hh / 2
    x2 = cx + w / 2
    y2 = cy + hh / 2

    conf = ch_ref[5] * obj
    cls = jnp.zeros((h, _W), f32)
    for i in range(1, nc):
        si = ch_ref[5 + i] * obj
        upd = si > conf
        conf = jnp.where(upd, si, conf)
        cls = jnp.where(upd, f32(i), cls)

    row_i = jax.lax.broadcasted_iota(jnp.int32, (h, _W), 0)
    col_i = jax.lax.broadcasted_iota(jnp.int32, (h, _W), 1)
    gidx = row_i * _W + col_i
    valid = (obj > _CONF_THRES) & (conf > _CONF_THRES) & (gidx < n)

    c = cls * _MAX_WH
    ox1 = x1 + c
    oy1 = y1 + c
    ox2 = x2 + c
    oy2 = y2 + c
    area = (ox2 - ox1) * (oy2 - oy1)
    s0 = jnp.where(valid, conf, neg_inf)

    lane = jax.lax.broadcasted_iota(jnp.int32, (_OUT_H, 8), 1)
    out_ref[...] = jnp.where(lane == 5, f32(-1.0), f32(0.0))

    def body(k, s):
        m = jnp.max(s)
        found = m > neg_inf
        idx = jnp.min(jnp.where(s == m, gidx, jnp.int32(h * _W)))
        pick = gidx == idx

        def ext(a):
            return jnp.sum(jnp.where(pick, a, f32(0.0)))

        wx1 = ext(x1)
        wy1 = ext(y1)
        wx2 = ext(x2)
        wy2 = ext(y2)
        wcls = ext(cls)
        wc = wcls * _MAX_WH
        wox1 = wx1 + wc
        woy1 = wy1 + wc
        wox2 = wx2 + wc
        woy2 = wy2 + wc
        warea = (wox2 - wox1) * (woy2 - woy1)

        xx1 = jnp.maximum(wox1, ox1)
        yy1 = jnp.maximum(woy1, oy1)
        xx2 = jnp.minimum(wox2, ox2)
        yy2 = jnp.minimum(woy2, oy2)
        inter = jnp.maximum(xx2 - xx1, f32(0.0)) * jnp.maximum(yy2 - yy1, f32(0.0))
        iou = inter / (warea + area - inter + f32(1e-12))
        s_new = jnp.where(found & (iou > _IOU_THRES), neg_inf, s)

        @pl.when(found)
        def _():
            lane1 = jax.lax.broadcasted_iota(jnp.int32, (1, 8), 1)
            row = jnp.where(lane1 == 0, wx1,
                  jnp.where(lane1 == 1, wy1,
                  jnp.where(lane1 == 2, wx2,
                  jnp.where(lane1 == 3, wy2,
                  jnp.where(lane1 == 4, m,
                  jnp.where(lane1 == 5, wcls, f32(0.0)))))))
            out_ref[pl.ds(k, 1), :] = row

        return s_new

    jax.lax.fori_loop(0, _MAX_DET, body, s0)


def kernel(prediction):
    x = prediction[0]
    n, chan = x.shape
    nc = chan - 5
    h = -(-n // _W)
    h = -(-h // 8) * 8
    np_ = h * _W
    xp = jnp.pad(x, ((0, np_ - n), (0, 0)))
    chans = xp.T.reshape(chan, h, _W)
    out = pl.pallas_call(
        functools.partial(_nms_kernel, n=n, nc=nc, h=h),
        out_shape=jax.ShapeDtypeStruct((_OUT_H, 8), jnp.float32),
    )(chans)
    return out[:_MAX_DET, :6]


# speedup vs baseline: 411.9838x; 1.0354x over previous
import functools

import jax
import jax.numpy as jnp
from jax.experimental import pallas as pl
from jax.experimental.pallas import tpu as pltpu

_CONF_THRES = 0.1
_IOU_THRES = 0.6
_MAX_WH = 4096.0
_MAX_DET = 300
_W = 128
_OUT_H = 304


def _nms_kernel(ch_ref, out_ref, sx1, sy1, sx2, sy2, scls, *, n, nc, h):
    f32 = jnp.float32
    neg_inf = f32(-jnp.inf)
    cx = ch_ref[0]
    cy = ch_ref[1]
    w = ch_ref[2]
    hh = ch_ref[3]
    obj = ch_ref[4]
    x1 = cx - w / 2
    y1 = cy - hh / 2
    x2 = cx + w / 2
    y2 = cy + hh / 2

    conf = ch_ref[5] * obj
    cls = jnp.zeros((h, _W), f32)
    for i in range(1, nc):
        si = ch_ref[5 + i] * obj
        upd = si > conf
        conf = jnp.where(upd, si, conf)
        cls = jnp.where(upd, f32(i), cls)

    row_i = jax.lax.broadcasted_iota(jnp.int32, (h, _W), 0)
    col_i = jax.lax.broadcasted_iota(jnp.int32, (h, _W), 1)
    gidx = row_i * _W + col_i
    valid = (obj > _CONF_THRES) & (conf > _CONF_THRES) & (gidx < n)

    c = cls * _MAX_WH
    ox1 = x1 + c
    oy1 = y1 + c
    ox2 = x2 + c
    oy2 = y2 + c
    area = (ox2 - ox1) * (oy2 - oy1)
    s0 = jnp.where(valid, conf, neg_inf)

    sx1[...] = x1
    sy1[...] = y1
    sx2[...] = x2
    sy2[...] = y2
    scls[...] = cls

    lane = jax.lax.broadcasted_iota(jnp.int32, (_OUT_H, 8), 1)
    out_ref[...] = jnp.where(lane == 5, f32(-1.0), f32(0.0))

    def body(k, s):
        m = jnp.max(s)
        found = m > neg_inf
        idx = jnp.min(jnp.where(s == m, gidx, jnp.int32(h * _W)))
        r = idx // _W
        lane_pick = jax.lax.broadcasted_iota(jnp.int32, (1, _W), 1) == (idx - r * _W)

        def ext(ref):
            return jnp.sum(jnp.where(lane_pick, ref[pl.ds(r, 1), :], f32(0.0)))

        wx1 = ext(sx1)
        wy1 = ext(sy1)
        wx2 = ext(sx2)
        wy2 = ext(sy2)
        wcls = ext(scls)
        wc = wcls * _MAX_WH
        wox1 = wx1 + wc
        woy1 = wy1 + wc
        wox2 = wx2 + wc
        woy2 = wy2 + wc
        warea = (wox2 - wox1) * (woy2 - woy1)

        xx1 = jnp.maximum(wox1, ox1)
        yy1 = jnp.maximum(woy1, oy1)
        xx2 = jnp.minimum(wox2, ox2)
        yy2 = jnp.minimum(woy2, oy2)
        inter = jnp.maximum(xx2 - xx1, f32(0.0)) * jnp.maximum(yy2 - yy1, f32(0.0))
        iou = inter / (warea + area - inter + f32(1e-12))
        s_new = jnp.where(found & (iou > _IOU_THRES), neg_inf, s)

        @pl.when(found)
        def _():
            lane1 = jax.lax.broadcasted_iota(jnp.int32, (1, 8), 1)
            row = jnp.where(lane1 == 0, wx1,
                  jnp.where(lane1 == 1, wy1,
                  jnp.where(lane1 == 2, wx2,
                  jnp.where(lane1 == 3, wy2,
                  jnp.where(lane1 == 4, m,
                  jnp.where(lane1 == 5, wcls, f32(0.0)))))))
            out_ref[pl.ds(k, 1), :] = row

        return s_new

    jax.lax.fori_loop(0, _MAX_DET, body, s0)


def kernel(prediction):
    x = prediction[0]
    n, chan = x.shape
    nc = chan - 5
    h = -(-n // _W)
    h = -(-h // 8) * 8
    np_ = h * _W
    xp = jnp.pad(x, ((0, np_ - n), (0, 0)))
    chans = xp.T.reshape(chan, h, _W)
    out = pl.pallas_call(
        functools.partial(_nms_kernel, n=n, nc=nc, h=h),
        out_shape=jax.ShapeDtypeStruct((_OUT_H, 8), jnp.float32),
        scratch_shapes=[pltpu.VMEM((h, _W), jnp.float32)] * 5,
    )(chans)
    return out[:_MAX_DET, :6]


# speedup vs baseline: 412.8523x; 1.0021x over previous
import functools

import jax
import jax.numpy as jnp
from jax.experimental import pallas as pl
from jax.experimental.pallas import tpu as pltpu

_CONF_THRES = 0.1
_IOU_THRES = 0.6
_MAX_WH = 4096.0
_MAX_DET = 300
_W = 128
_OUT_H = 304


def _nms_kernel(ch_ref, out_ref, sx1, sy1, sx2, sy2, scls, *, n, nc, h):
    f32 = jnp.float32
    neg_inf = f32(-jnp.inf)
    cx = ch_ref[0]
    cy = ch_ref[1]
    w = ch_ref[2]
    hh = ch_ref[3]
    obj = ch_ref[4]
    x1 = cx - w / 2
    y1 = cy - hh / 2
    x2 = cx + w / 2
    y2 = cy + hh / 2

    conf = ch_ref[5] * obj
    cls = jnp.zeros((h, _W), f32)
    for i in range(1, nc):
        si = ch_ref[5 + i] * obj
        upd = si > conf
        conf = jnp.where(upd, si, conf)
        cls = jnp.where(upd, f32(i), cls)

    row_i = jax.lax.broadcasted_iota(jnp.int32, (h, _W), 0)
    col_i = jax.lax.broadcasted_iota(jnp.int32, (h, _W), 1)
    gidx = row_i * _W + col_i
    valid = (obj > _CONF_THRES) & (conf > _CONF_THRES) & (gidx < n)

    c = cls * _MAX_WH
    ox1 = x1 + c
    oy1 = y1 + c
    ox2 = x2 + c
    oy2 = y2 + c
    area = (ox2 - ox1) * (oy2 - oy1)
    s0 = jnp.where(valid, conf, neg_inf)

    sx1[...] = x1
    sy1[...] = y1
    sx2[...] = x2
    sy2[...] = y2
    scls[...] = cls

    lane = jax.lax.broadcasted_iota(jnp.int32, (_OUT_H, 8), 1)
    out_ref[...] = jnp.where(lane == 5, f32(-1.0), f32(0.0))

    nb = h // 8
    big = jnp.int32(h * _W)

    def argmax_tail(acc_v, acc_i):
        m = jnp.max(acc_v)
        idx = jnp.min(jnp.where(acc_v == m, acc_i, big))
        return m, jnp.minimum(idx, big - 1)

    def body(k, carry):
        s, m, idx = carry
        found = m > neg_inf
        r = idx // _W
        lane_pick = jax.lax.broadcasted_iota(jnp.int32, (1, _W), 1) == (idx - r * _W)

        def ext(ref):
            return jnp.sum(jnp.where(lane_pick, ref[pl.ds(r, 1), :], f32(0.0)))

        wx1 = ext(sx1)
        wy1 = ext(sy1)
        wx2 = ext(sx2)
        wy2 = ext(sy2)
        wcls = ext(scls)
        wc = wcls * _MAX_WH
        wox1 = wx1 + wc
        woy1 = wy1 + wc
        wox2 = wx2 + wc
        woy2 = wy2 + wc
        warea = (wox2 - wox1) * (woy2 - woy1)

        acc_v = jnp.full((8, _W), -jnp.inf, f32)
        acc_i = jnp.full((8, _W), big, jnp.int32)
        parts = []
        for b in range(nb):
            sl = slice(8 * b, 8 * (b + 1))
            xx1 = jnp.maximum(wox1, ox1[sl])
            yy1 = jnp.maximum(woy1, oy1[sl])
            xx2 = jnp.minimum(wox2, ox2[sl])
            yy2 = jnp.minimum(woy2, oy2[sl])
            inter = jnp.maximum(xx2 - xx1, f32(0.0)) * jnp.maximum(yy2 - yy1, f32(0.0))
            iou = inter / (warea + area[sl] - inter + f32(1e-12))
            sb = jnp.where(found & (iou > _IOU_THRES), neg_inf, s[sl])
            upd = sb > acc_v
            acc_v = jnp.where(upd, sb, acc_v)
            acc_i = jnp.where(upd, gidx[sl], acc_i)
            parts.append(sb)
        s_new = jnp.concatenate(parts, axis=0)
        m2, idx2 = argmax_tail(acc_v, acc_i)

        @pl.when(found)
        def _():
            lane1 = jax.lax.broadcasted_iota(jnp.int32, (1, 8), 1)
            row = jnp.where(lane1 == 0, wx1,
                  jnp.where(lane1 == 1, wy1,
                  jnp.where(lane1 == 2, wx2,
                  jnp.where(lane1 == 3, wy2,
                  jnp.where(lane1 == 4, m,
                  jnp.where(lane1 == 5, wcls, f32(0.0)))))))
            out_ref[pl.ds(k, 1), :] = row

        return s_new, m2, idx2

    acc_v = jnp.full((8, _W), -jnp.inf, f32)
    acc_i = jnp.full((8, _W), big, jnp.int32)
    for b in range(nb):
        sl = slice(8 * b, 8 * (b + 1))
        sb = s0[sl]
        upd = sb > acc_v
        acc_v = jnp.where(upd, sb, acc_v)
        acc_i = jnp.where(upd, gidx[sl], acc_i)
    m0, idx0 = argmax_tail(acc_v, acc_i)

    jax.lax.fori_loop(0, _MAX_DET, body, (s0, m0, idx0))


def kernel(prediction):
    x = prediction[0]
    n, chan = x.shape
    nc = chan - 5
    h = -(-n // _W)
    h = -(-h // 8) * 8
    np_ = h * _W
    xp = jnp.pad(x, ((0, np_ - n), (0, 0)))
    chans = xp.T.reshape(chan, h, _W)
    out = pl.pallas_call(
        functools.partial(_nms_kernel, n=n, nc=nc, h=h),
        out_shape=jax.ShapeDtypeStruct((_OUT_H, 8), jnp.float32),
        scratch_shapes=[pltpu.VMEM((h, _W), jnp.float32)] * 5,
    )(chans)
    return out[:_MAX_DET, :6]


# speedup vs baseline: 414.2333x; 1.0033x over previous
import functools

import jax
import jax.numpy as jnp
from jax.experimental import pallas as pl
from jax.experimental.pallas import tpu as pltpu

_CONF_THRES = 0.1
_IOU_THRES = 0.6
_MAX_WH = 4096.0
_MAX_DET = 300
_W = 128
_OUT_H = 304


def _nms_kernel(ch_ref, out_ref, sx1, sy1, sx2, sy2, scls, *, n, nc, h):
    f32 = jnp.float32
    neg_inf = f32(-jnp.inf)
    cx = ch_ref[0]
    cy = ch_ref[1]
    w = ch_ref[2]
    hh = ch_ref[3]
    obj = ch_ref[4]
    x1 = cx - w / 2
    y1 = cy - hh / 2
    x2 = cx + w / 2
    y2 = cy + hh / 2

    conf = ch_ref[5] * obj
    cls = jnp.zeros((h, _W), f32)
    for i in range(1, nc):
        si = ch_ref[5 + i] * obj
        upd = si > conf
        conf = jnp.where(upd, si, conf)
        cls = jnp.where(upd, f32(i), cls)

    row_i = jax.lax.broadcasted_iota(jnp.int32, (h, _W), 0)
    col_i = jax.lax.broadcasted_iota(jnp.int32, (h, _W), 1)
    gidx = row_i * _W + col_i
    valid = (obj > _CONF_THRES) & (conf > _CONF_THRES) & (gidx < n)

    c = cls * _MAX_WH
    ox1 = x1 + c
    oy1 = y1 + c
    ox2 = x2 + c
    oy2 = y2 + c
    area = (ox2 - ox1) * (oy2 - oy1)
    s0 = jnp.where(valid, conf, neg_inf)

    sx1[...] = x1
    sy1[...] = y1
    sx2[...] = x2
    sy2[...] = y2
    scls[...] = cls

    lane = jax.lax.broadcasted_iota(jnp.int32, (_OUT_H, 8), 1)
    out_ref[...] = jnp.where(lane == 5, f32(-1.0), f32(0.0))

    nb = h // 8
    big = jnp.int32(h * _W)

    def argmax_tail(acc_v, acc_i):
        m = jnp.max(acc_v)
        idx = jnp.min(jnp.where(acc_v == m, acc_i, big))
        return m, jnp.minimum(idx, big - 1)

    def body(k, carry):
        s, m, idx = carry
        found = m > neg_inf
        r = idx // _W
        lane_pick = jax.lax.broadcasted_iota(jnp.int32, (1, _W), 1) == (idx - r * _W)

        def ext(ref):
            return jnp.sum(jnp.where(lane_pick, ref[pl.ds(r, 1), :], f32(0.0)))

        wx1 = ext(sx1)
        wy1 = ext(sy1)
        wx2 = ext(sx2)
        wy2 = ext(sy2)
        wcls = ext(scls)
        wc = wcls * _MAX_WH
        wox1 = wx1 + wc
        woy1 = wy1 + wc
        wox2 = wx2 + wc
        woy2 = wy2 + wc
        warea = (wox2 - wox1) * (woy2 - woy1)

        acc_v = jnp.full((8, _W), -jnp.inf, f32)
        acc_i = jnp.full((8, _W), big, jnp.int32)
        parts = []
        for b in range(nb):
            sl = slice(8 * b, 8 * (b + 1))
            ox1b = ox1[sl]
            oy1b = oy1[sl]
            ox2b = ox2[sl]
            oy2b = oy2[sl]
            xx1 = jnp.maximum(wox1, ox1b)
            yy1 = jnp.maximum(woy1, oy1b)
            xx2 = jnp.minimum(wox2, ox2b)
            yy2 = jnp.minimum(woy2, oy2b)
            inter = jnp.maximum(xx2 - xx1, f32(0.0)) * jnp.maximum(yy2 - yy1, f32(0.0))
            area_b = (ox2b - ox1b) * (oy2b - oy1b)
            iou = inter / (warea + area_b - inter + f32(1e-12))
            sb = jnp.where(found & (iou > _IOU_THRES), neg_inf, s[sl])
            upd = sb > acc_v
            acc_v = jnp.where(upd, sb, acc_v)
            acc_i = jnp.where(upd, gidx[sl], acc_i)
            parts.append(sb)
        s_new = jnp.concatenate(parts, axis=0)
        m2, idx2 = argmax_tail(acc_v, acc_i)

        @pl.when(found)
        def _():
            lane1 = jax.lax.broadcasted_iota(jnp.int32, (1, 8), 1)
            row = jnp.where(lane1 == 0, wx1,
                  jnp.where(lane1 == 1, wy1,
                  jnp.where(lane1 == 2, wx2,
                  jnp.where(lane1 == 3, wy2,
                  jnp.where(lane1 == 4, m,
                  jnp.where(lane1 == 5, wcls, f32(0.0)))))))
            out_ref[pl.ds(k, 1), :] = row

        return s_new, m2, idx2

    acc_v = jnp.full((8, _W), -jnp.inf, f32)
    acc_i = jnp.full((8, _W), big, jnp.int32)
    for b in range(nb):
        sl = slice(8 * b, 8 * (b + 1))
        sb = s0[sl]
        upd = sb > acc_v
        acc_v = jnp.where(upd, sb, acc_v)
        acc_i = jnp.where(upd, gidx[sl], acc_i)
    m0, idx0 = argmax_tail(acc_v, acc_i)

    jax.lax.fori_loop(0, _MAX_DET, body, (s0, m0, idx0))


def kernel(prediction):
    x = prediction[0]
    n, chan = x.shape
    nc = chan - 5
    h = -(-n // _W)
    h = -(-h // 8) * 8
    np_ = h * _W
    xp = jnp.pad(x, ((0, np_ - n), (0, 0)))
    chans = xp.T.reshape(chan, h, _W)
    out = pl.pallas_call(
        functools.partial(_nms_kernel, n=n, nc=nc, h=h),
        out_shape=jax.ShapeDtypeStruct((_OUT_H, 8), jnp.float32),
        scratch_shapes=[pltpu.VMEM((h, _W), jnp.float32)] * 5,
    )(chans)
    return out[:_MAX_DET, :6]
